# Initial kernel scaffold; baseline (speedup 1.0000x reference)
import jax
import jax.numpy as jnp
from jax.experimental import pallas as pl


def kernel(x, tables):
    raise NotImplementedError("write your pallas kernel here")


# speedup vs baseline: 1.1650x; 1.1650x over previous
import functools

import jax
import jax.numpy as jnp
from jax import lax
from jax.experimental import pallas as pl
from jax.experimental.pallas import tpu as pltpu
from jax.experimental.pallas import tpu_sc as plsc

NUM_INPUTS = 26
STATE_SIZE = 32
CARDINALITY = 100000
BATCH = 16384

_info = plsc.get_sparse_core_info()
NC, NS, L = _info.num_cores, _info.num_subcores, _info.num_lanes
NW = NC * NS
B_PER_W = BATCH // NW
CHUNK = 128
NCHUNK = B_PER_W // CHUNK


@functools.partial(
    pl.kernel,
    mesh=plsc.VectorSubcoreMesh(core_axis_name="c", subcore_axis_name="s"),
    out_type=jax.ShapeDtypeStruct((NUM_INPUTS, BATCH, STATE_SIZE), jnp.float32),
    scratch_types=[
        pltpu.VMEM((NCHUNK, CHUNK), jnp.int32),
        pltpu.VMEM((B_PER_W, STATE_SIZE), jnp.float32),
        pltpu.SemaphoreType.DMA,
    ],
    compiler_params=pltpu.CompilerParams(use_tc_tiling_on_sc=False),
)
def _sc_gather(idx4_hbm, tab_hbm, out_hbm, idx_v, rows_v, sem):
    wid = lax.axis_index("s") * NC + lax.axis_index("c")
    base = wid * B_PER_W

    def body(i, _):
        pltpu.sync_copy(idx4_hbm.at[i, wid], idx_v)
        off = jnp.full((L,), i * CARDINALITY, dtype=jnp.int32)
        for k in range(NCHUNK):
            for j in range(CHUNK // L):
                sl = pl.ds(j * L, L)
                idx_v[k, sl] = idx_v[k, sl] + off
        cps = [
            pltpu.async_copy(
                tab_hbm.at[idx_v.at[k]],
                rows_v.at[pl.ds(k * CHUNK, CHUNK)],
                sem,
            )
            for k in range(NCHUNK)
        ]
        for c in cps:
            c.wait()
        pltpu.sync_copy(rows_v, out_hbm.at[i, pl.ds(base, B_PER_W)])
        return _

    lax.fori_loop(0, NUM_INPUTS, body, None)


def kernel(x, tables):
    idx4 = x.T.astype(jnp.int32).reshape(NUM_INPUTS, NW, NCHUNK, CHUNK)
    tab = tables.reshape(NUM_INPUTS * CARDINALITY, STATE_SIZE)
    return _sc_gather(idx4, tab)
